# baseline (speedup 1.0000x reference)
import functools
import math

import jax
import jax.numpy as jnp
from jax import lax
from jax.experimental import pallas as pl
from jax.experimental.pallas import tpu as pltpu
from jax.experimental.pallas import tpu_sc as plsc

NUM_TOKENS = 8192
INPUT_DIM = 1024
NUM_EXPERTS = 64
TOP_K = 8
CAPACITY = 2048
HIDDEN_DIM = 64

TILE = 1024
NUM_TILES = NUM_TOKENS // TILE

NW = 32
TOK_PER_W = NUM_TOKENS // NW


def _logits_body(x_ref, gwt_ref, gb_ref, out_ref):
    out_ref[...] = (
        jnp.dot(x_ref[...], gwt_ref[...], preferred_element_type=jnp.float32)
        + gb_ref[...]
    )


def _tree(op, xs):
    xs = list(xs)
    while len(xs) > 1:
        nxt = []
        for i in range(0, len(xs), 2):
            nxt.append(op(xs[i], xs[i + 1]) if i + 1 < len(xs) else xs[i])
        xs = nxt
    return xs[0]


def _sc_route_body(lt_hbm, wgtT_hbm, lv, ov):
    wid = lax.axis_index("s") * 2 + lax.axis_index("c")
    base = wid * TOK_PER_W
    pltpu.sync_copy(lt_hbm.at[:, pl.ds(base, TOK_PER_W)], lv)
    lane = lax.iota(jnp.int32, 16)
    neg = jnp.float32(-3.0e38)
    E = NUM_EXPERTS

    def group(g, carry):
        s = g * 16
        v = [lv[j, pl.ds(s, 16)] for j in range(E)]
        gmax = _tree(jnp.maximum, v)
        e = [jnp.exp(vj - gmax) for vj in v]
        rs = 1.0 / _tree(jnp.add, e)
        cur = list(v)
        selb = [lane < 0 for _ in range(E)]
        for _ in range(TOP_K):
            m = _tree(jnp.maximum, cur)
            am = _tree(
                jnp.minimum,
                [
                    jnp.where(cur[j] == m, jnp.int32(j), jnp.int32(E))
                    for j in range(E)
                ],
            )
            for j in range(E):
                hit = am == jnp.int32(j)
                selb[j] = selb[j] | hit
                cur[j] = jnp.where(hit, neg, cur[j])
        for j in range(E):
            ov[j, pl.ds(s, 16)] = jnp.where(selb[j], e[j] * rs, 0.0)
        return carry

    lax.fori_loop(0, TOK_PER_W // 16, group, 0)
    pltpu.sync_copy(ov, wgtT_hbm.at[:, pl.ds(base, TOK_PER_W)])


def _moe_body(
    x_ref, wsel_ref, trilt_ref,
    w1c_ref, b1c_ref, w2sp_ref, expb_ref, out_ref, cnt_ref,
):
    i = pl.program_id(0)

    @pl.when(i == 0)
    def _():
        cnt_ref[...] = jnp.zeros_like(cnt_ref)

    E, H = NUM_EXPERTS, HIDDEN_DIM
    xf = x_ref[...]
    wsel = wsel_ref[...]
    sel = (wsel > 0.0).astype(jnp.float32)

    run = cnt_ref[...]
    cs_tok = jnp.dot(
        trilt_ref[...], sel.astype(jnp.bfloat16), preferred_element_type=jnp.float32
    )
    keep = ((run + cs_tok - 1.0) < float(CAPACITY)).astype(jnp.float32)
    wgt = wsel * keep
    cnt_ref[...] = run + jnp.sum(sel, axis=0, keepdims=True)

    h = (
        jnp.dot(
            xf.astype(jnp.bfloat16), w1c_ref[...],
            preferred_element_type=jnp.float32,
        )
        + b1c_ref[...]
    )
    hg = h * (0.5 * jax.lax.erf(h * (1.0 / math.sqrt(2.0))) + 0.5)
    we = jnp.dot(
        wgt.astype(jnp.bfloat16), expb_ref[...], preferred_element_type=jnp.float32
    )
    g = (hg * we[:, : E * H]).astype(jnp.bfloat16)
    acc = jnp.dot(g, w2sp_ref[...], preferred_element_type=jnp.float32)
    out_ref[...] = acc[:, :H] + we[:, E * H :]


@jax.jit
def kernel(x, gate_w, gate_b, w1, b1, w2, b2):
    E, H, D = NUM_EXPERTS, HIDDEN_DIM, INPUT_DIM

    gwt = gate_w.T
    gb = gate_b.reshape(1, E)
    w1c = jnp.transpose(w1, (2, 0, 1)).reshape(D, E * H).astype(jnp.bfloat16)
    b1c = b1.reshape(1, E * H)
    w2s = jnp.transpose(w2, (0, 2, 1)).reshape(E * H, H)
    w2sp = (
        jnp.concatenate([w2s, jnp.zeros((E * H, 128 - H), jnp.float32)], axis=1)
        .astype(jnp.bfloat16)
    )
    trilt = jnp.tril(jnp.ones((TILE, TILE), jnp.bfloat16))
    expand = jnp.repeat(jnp.eye(E, dtype=jnp.bfloat16), H, axis=1)
    expb = jnp.concatenate([expand, b2.astype(jnp.bfloat16)], axis=1)

    logits = pl.pallas_call(
        _logits_body,
        grid=(NUM_TILES,),
        in_specs=[
            pl.BlockSpec((TILE, D), lambda i: (i, 0)),
            pl.BlockSpec((D, E), lambda i: (0, 0)),
            pl.BlockSpec((1, E), lambda i: (0, 0)),
        ],
        out_specs=pl.BlockSpec((TILE, E), lambda i: (i, 0)),
        out_shape=jax.ShapeDtypeStruct((NUM_TOKENS, E), jnp.float32),
    )(x, gwt, gb)

    logits_t = logits.T

    sc_route = functools.partial(
        pl.kernel,
        mesh=plsc.VectorSubcoreMesh(core_axis_name="c", subcore_axis_name="s"),
        out_type=jax.ShapeDtypeStruct((E, NUM_TOKENS), jnp.float32),
        scratch_types=[
            pltpu.VMEM((E, TOK_PER_W), jnp.float32),
            pltpu.VMEM((E, TOK_PER_W), jnp.float32),
        ],
    )(_sc_route_body)
    wsel = sc_route(logits_t).T

    out = pl.pallas_call(
        _moe_body,
        grid=(NUM_TILES,),
        in_specs=[
            pl.BlockSpec((TILE, D), lambda i: (i, 0)),
            pl.BlockSpec((TILE, E), lambda i: (i, 0)),
            pl.BlockSpec((TILE, TILE), lambda i: (0, 0)),
            pl.BlockSpec((D, E * H), lambda i: (0, 0)),
            pl.BlockSpec((1, E * H), lambda i: (0, 0)),
            pl.BlockSpec((E * H, 128), lambda i: (0, 0)),
            pl.BlockSpec((E, E * H + H), lambda i: (0, 0)),
        ],
        out_specs=pl.BlockSpec((TILE, H), lambda i: (i, 0)),
        out_shape=jax.ShapeDtypeStruct((NUM_TOKENS, H), jnp.float32),
        scratch_shapes=[pltpu.VMEM((1, E), jnp.float32)],
    )(x, wsel, trilt, w1c, b1c, w2sp, expb)

    return out, jnp.float32(0.0)


# speedup vs baseline: 1.2492x; 1.2492x over previous
import math

import jax
import jax.numpy as jnp
from jax.experimental import pallas as pl
from jax.experimental.pallas import tpu as pltpu

NUM_TOKENS = 8192
INPUT_DIM = 1024
NUM_EXPERTS = 64
TOP_K = 8
CAPACITY = 2048
HIDDEN_DIM = 64

TILE = 1024
NUM_TILES = NUM_TOKENS // TILE


def _moe_body(
    x_ref, gwt_ref, gb_ref, trile_ref, trilt_ref,
    w1c_ref, b1c_ref, w2sp_ref, expb_ref, out_ref, cnt_ref,
):
    i = pl.program_id(0)

    @pl.when(i == 0)
    def _():
        cnt_ref[...] = jnp.zeros_like(cnt_ref)

    E, H = NUM_EXPERTS, HIDDEN_DIM
    xf = x_ref[...]

    logits = (
        jnp.dot(xf, gwt_ref[...], preferred_element_type=jnp.float32)
        + gb_ref[...]
    )
    z = logits - jnp.max(logits, axis=1, keepdims=True)
    ez = jnp.exp(z)
    probs = ez / jnp.sum(ez, axis=1, keepdims=True)

    sel = jnp.zeros((TILE, E), jnp.float32)
    lcur = logits
    neg = jnp.float32(-3.0e38)
    for _ in range(TOP_K):
        m = jnp.max(lcur, axis=1, keepdims=True)
        oh = (lcur == m).astype(jnp.float32)
        cs = jnp.dot(oh.astype(jnp.bfloat16), trile_ref[...], preferred_element_type=jnp.float32)
        first = oh * (cs == 1.0).astype(jnp.float32)
        sel = sel + first
        lcur = jnp.where(first > 0.0, neg, lcur)

    run = cnt_ref[...]
    cs_tok = jnp.dot(trilt_ref[...], sel.astype(jnp.bfloat16), preferred_element_type=jnp.float32)
    keep = ((run + cs_tok - 1.0) < float(CAPACITY)).astype(jnp.float32)
    wgt = probs * sel * keep
    cnt_ref[...] = run + jnp.sum(sel, axis=0, keepdims=True)

    h = (
        jnp.dot(
            xf.astype(jnp.bfloat16), w1c_ref[...],
            preferred_element_type=jnp.float32,
        )
        + b1c_ref[...]
    )
    half = 0.5 * h
    hg = half * jax.lax.erf(h * (1.0 / math.sqrt(2.0))) + half
    we = jnp.dot(wgt.astype(jnp.bfloat16), expb_ref[...], preferred_element_type=jnp.float32)
    g = (hg * we[:, : E * H]).astype(jnp.bfloat16)
    acc = jnp.dot(g, w2sp_ref[...], preferred_element_type=jnp.float32)
    out_ref[...] = acc[:, :H] + we[:, E * H :]


@jax.jit
def kernel(x, gate_w, gate_b, w1, b1, w2, b2):
    E, H, D = NUM_EXPERTS, HIDDEN_DIM, INPUT_DIM

    gwt = gate_w.T
    gb = gate_b.reshape(1, E)
    w1c = jnp.transpose(w1, (2, 0, 1)).reshape(D, E * H).astype(jnp.bfloat16)
    b1c = b1.reshape(1, E * H)
    w2s = jnp.transpose(w2, (0, 2, 1)).reshape(E * H, H)
    w2sp = (
        jnp.concatenate([w2s, jnp.zeros((E * H, 128 - H), jnp.float32)], axis=1)
        .astype(jnp.bfloat16)
    )
    trile = jnp.triu(jnp.ones((E, E), jnp.bfloat16))
    trilt = jnp.tril(jnp.ones((TILE, TILE), jnp.bfloat16))
    expand = jnp.repeat(jnp.eye(E, dtype=jnp.bfloat16), H, axis=1)
    expb = jnp.concatenate([expand, b2.astype(jnp.bfloat16)], axis=1)

    out = pl.pallas_call(
        _moe_body,
        grid=(NUM_TILES,),
        in_specs=[
            pl.BlockSpec((TILE, D), lambda i: (i, 0)),
            pl.BlockSpec((D, E), lambda i: (0, 0)),
            pl.BlockSpec((1, E), lambda i: (0, 0)),
            pl.BlockSpec((E, E), lambda i: (0, 0)),
            pl.BlockSpec((TILE, TILE), lambda i: (0, 0)),
            pl.BlockSpec((D, E * H), lambda i: (0, 0)),
            pl.BlockSpec((1, E * H), lambda i: (0, 0)),
            pl.BlockSpec((E * H, 128), lambda i: (0, 0)),
            pl.BlockSpec((E, E * H + H), lambda i: (0, 0)),
        ],
        out_specs=pl.BlockSpec((TILE, H), lambda i: (i, 0)),
        out_shape=jax.ShapeDtypeStruct((NUM_TOKENS, H), jnp.float32),
        scratch_shapes=[pltpu.VMEM((1, E), jnp.float32)],
    )(x, gwt, gb, trile, trilt, w1c, b1c, w2sp, expb)

    return out, jnp.float32(0.0)
